# Initial kernel scaffold; baseline (speedup 1.0000x reference)
import jax
import jax.numpy as jnp
from jax.experimental import pallas as pl


def kernel(pred_cls, pred_box, anchors, mask, gt_labels, gt_bboxes):
    raise NotImplementedError("write your pallas kernel here")


# speedup vs baseline: 34.8288x; 34.8288x over previous
import functools

import jax
import jax.numpy as jnp
from jax.experimental import pallas as pl

NUM_CLASSES = 80
ALPHA = 0.25
GAMMA = 2.0
TOPK = 10
BIGI = 10 ** 9
REMOVED = 3.0e38


def _softplus_neg_abs(x):
    return jnp.log(1.0 + jnp.exp(-jnp.abs(x)))


def _loss0(x):
    p = 1.0 / (1.0 + jnp.exp(-x))
    ce = jnp.maximum(x, 0.0) + _softplus_neg_abs(x)
    return (1.0 - ALPHA) * ce * p * p


def _loss1(x):
    p = 1.0 / (1.0 + jnp.exp(-x))
    ce = jnp.maximum(x, 0.0) - x + _softplus_neg_abs(x)
    q = 1.0 - p
    return ALPHA * ce * q * q


def _ota_kernel(pred_cls_ref, pbt_ref, anc_ref, mask_ref, glab_ref,
                gbox_ref, cls_ref, reg_ref, npos_ref):
    M = pbt_ref.shape[2]
    G = gbox_ref.shape[1]

    x_cls = pred_cls_ref[0]
    pbt = pbt_ref[0]
    anc = anc_ref[...]
    inv_mask = 1.0 - mask_ref[0]
    glab = glab_ref[0]
    gbox = gbox_ref[0]

    iota_m = jax.lax.broadcasted_iota(jnp.int32, (1, M), 1)
    iota_g = jax.lax.broadcasted_iota(jnp.int32, (G, 1), 0)

    ax = anc[0:1, :]
    ay = anc[1:2, :]
    px1, py1, px2, py2 = (pbt[0:1, :], pbt[1:2, :], pbt[2:3, :], pbt[3:4, :])
    gx1, gy1, gx2, gy2 = (gbox[:, 0:1], gbox[:, 1:2], gbox[:, 2:3], gbox[:, 3:4])

    d = jnp.minimum(jnp.minimum(ax - gx1, ay - gy1),
                    jnp.minimum(gx2 - ax, gy2 - ay))
    is_in_gt = d > 0.0
    valid_any = jnp.max(jnp.where(is_in_gt, 1.0, 0.0), axis=0, keepdims=True)

    area_g = jnp.clip(gx2 - gx1, 0.0) * jnp.clip(gy2 - gy1, 0.0)
    area_p = jnp.clip(px2 - px1, 0.0) * jnp.clip(py2 - py1, 0.0)
    iw = jnp.clip(jnp.minimum(gx2, px2) - jnp.maximum(gx1, px1), 0.0)
    ih = jnp.clip(jnp.minimum(gy2, py2) - jnp.maximum(gy1, py1), 0.0)
    inter = iw * ih
    union = area_g + area_p - inter
    ious = inter / jnp.clip(union, 1e-8)

    iota_c = jax.lax.broadcasted_iota(jnp.int32, (G, NUM_CLASSES), 1)
    onehot_lab = (iota_c == glab).astype(jnp.float32)
    logits_lab = jax.lax.dot_general(
        onehot_lab, x_cls, (((1,), (1,)), ((), ())),
        preferred_element_type=jnp.float32)

    p_lab = 1.0 / (1.0 + jnp.exp(-logits_lab))
    sf = (ious - p_lab) * (ious - p_lab)
    ce = (jnp.maximum(logits_lab, 0.0) - logits_lab * ious
          + _softplus_neg_abs(logits_lab))
    cost = ce * sf - 3.0 * jnp.log(ious + 1e-8)
    cost = jnp.where(is_in_gt & (valid_any > 0.0), cost, cost + 1e8)

    work = ious
    s10 = jnp.zeros((G, 1), jnp.float32)
    for _ in range(TOPK):
        vmax = jnp.max(work, axis=1, keepdims=True)
        s10 = s10 + vmax
        imax = jnp.min(jnp.where(work == vmax, iota_m, BIGI),
                       axis=1, keepdims=True)
        work = jnp.where(iota_m == imax, -1.0, work)
    ks = jnp.clip(jnp.floor(s10), 1.0, float(M))

    work = cost
    match = jnp.zeros((G, M), jnp.float32)
    for i in range(TOPK):
        vmin = jnp.min(work, axis=1, keepdims=True)
        imin = jnp.min(jnp.where(work == vmin, iota_m, BIGI),
                       axis=1, keepdims=True)
        sel = iota_m == imin
        take = jnp.where(sel & (float(i) < ks), 1.0, 0.0)
        match = match + take
        work = jnp.where(sel, REMOVED, work)

    n_match = jnp.sum(match, axis=0, keepdims=True)
    cmin = jnp.min(cost, axis=0, keepdims=True)
    gmin = jnp.min(jnp.where(cost == cmin, iota_g, BIGI),
                   axis=0, keepdims=True)
    keep = (iota_g == gmin).astype(jnp.float32)
    match = jnp.where(n_match > 1.0, keep, match)

    fg = jnp.max(match, axis=0, keepdims=True)
    gfirst = jnp.min(jnp.where(match > 0.0, iota_g, BIGI),
                     axis=0, keepdims=True)
    sel_one = (iota_g == gfirst).astype(jnp.float32)

    s_bg = jnp.sum(_loss0(x_cls), axis=1, keepdims=True)
    bg_sum = jax.lax.dot_general(
        inv_mask, s_bg, (((1,), (0,)), ((), ())),
        preferred_element_type=jnp.float32)
    corr = jnp.sum(sel_one * (_loss1(logits_lab) - _loss0(logits_lab)),
                   axis=0, keepdims=True)
    cls_sum = bg_sum + jnp.sum(inv_mask * corr, axis=1, keepdims=True)

    tx1 = jnp.sum(sel_one * gx1, axis=0, keepdims=True)
    ty1 = jnp.sum(sel_one * gy1, axis=0, keepdims=True)
    tx2 = jnp.sum(sel_one * gx2, axis=0, keepdims=True)
    ty2 = jnp.sum(sel_one * gy2, axis=0, keepdims=True)
    a2 = jnp.clip(tx2 - tx1, 0.0) * jnp.clip(ty2 - ty1, 0.0)
    iw = jnp.clip(jnp.minimum(px2, tx2) - jnp.maximum(px1, tx1), 0.0)
    ih = jnp.clip(jnp.minimum(py2, ty2) - jnp.maximum(py1, ty1), 0.0)
    inter = iw * ih
    union = area_p + a2 - inter
    iou = inter / jnp.clip(union, 1e-8)
    cw = jnp.clip(jnp.maximum(px2, tx2) - jnp.minimum(px1, tx1), 0.0)
    ch = jnp.clip(jnp.maximum(py2, ty2) - jnp.minimum(py1, ty1), 0.0)
    carea = cw * ch
    gi = iou - (carea - union) / jnp.clip(carea, 1e-8)
    reg_sum = jnp.sum(fg * (1.0 - gi), axis=1, keepdims=True)

    cls_ref[0] = cls_sum
    reg_ref[0] = reg_sum
    npos_ref[0] = jnp.sum(fg, axis=1, keepdims=True)


@jax.jit
def kernel(pred_cls, pred_box, anchors, mask, gt_labels, gt_bboxes):
    B, M, C = pred_cls.shape
    G = gt_bboxes.shape[1]

    pbt = jnp.transpose(pred_box, (0, 2, 1))
    anc = jnp.transpose(anchors[:, :2], (1, 0))
    mask_f = mask.astype(jnp.float32).reshape(B, 1, M)
    glab = gt_labels.astype(jnp.int32).reshape(B, G, 1)

    out_sd = jax.ShapeDtypeStruct((B, 1, 1), jnp.float32)
    cls_s, reg_s, npos = pl.pallas_call(
        _ota_kernel,
        grid=(B,),
        in_specs=[
            pl.BlockSpec((1, M, C), lambda b: (b, 0, 0)),
            pl.BlockSpec((1, 4, M), lambda b: (b, 0, 0)),
            pl.BlockSpec((2, M), lambda b: (0, 0)),
            pl.BlockSpec((1, 1, M), lambda b: (b, 0, 0)),
            pl.BlockSpec((1, G, 1), lambda b: (b, 0, 0)),
            pl.BlockSpec((1, G, 4), lambda b: (b, 0, 0)),
        ],
        out_specs=[
            pl.BlockSpec((1, 1, 1), lambda b: (b, 0, 0)),
            pl.BlockSpec((1, 1, 1), lambda b: (b, 0, 0)),
            pl.BlockSpec((1, 1, 1), lambda b: (b, 0, 0)),
        ],
        out_shape=[out_sd, out_sd, out_sd],
    )(pred_cls, pbt, anc, mask_f, glab, gt_bboxes)

    num_fgs = jnp.maximum(jnp.sum(npos), 1.0)
    return jnp.sum(cls_s) / num_fgs, jnp.sum(reg_s) / num_fgs


# speedup vs baseline: 44.6510x; 1.2820x over previous
import jax
import jax.numpy as jnp
from jax.experimental import pallas as pl

NUM_CLASSES = 80
ALPHA = 0.25
TOPK = 10
BIGI = 10 ** 9
REMOVED = 3.0e38


def _dot(a, b):
    return jax.lax.dot_general(a, b, (((1,), (0,)), ((), ())),
                               preferred_element_type=jnp.float32)


def _ota_kernel(pred_cls_ref, pbt_ref, anc_ref, mask_ref, glab_ref,
                gbox_ref, cls_ref, reg_ref, npos_ref):
    M = pbt_ref.shape[2]
    G = gbox_ref.shape[1]

    x_cls = pred_cls_ref[0]
    pbt = pbt_ref[0]
    anc = anc_ref[...]
    inv_mask = 1.0 - mask_ref[0]
    glab = glab_ref[0]
    gbox = gbox_ref[0]

    iota_m = jax.lax.broadcasted_iota(jnp.int32, (1, M), 1)
    iota_g = jax.lax.broadcasted_iota(jnp.int32, (G, 1), 0)
    ones_m = jnp.ones((M, 1), jnp.float32)

    ax = anc[0:1, :]
    ay = anc[1:2, :]
    px1, py1, px2, py2 = (pbt[0:1, :], pbt[1:2, :], pbt[2:3, :], pbt[3:4, :])
    gx1, gy1, gx2, gy2 = (gbox[:, 0:1], gbox[:, 1:2], gbox[:, 2:3], gbox[:, 3:4])

    d = jnp.minimum(jnp.minimum(ax - gx1, ay - gy1),
                    jnp.minimum(gx2 - ax, gy2 - ay))
    is_in_gt = d > 0.0
    valid_any = jnp.max(jnp.where(is_in_gt, 1.0, 0.0), axis=0, keepdims=True)

    area_g = jnp.clip(gx2 - gx1, 0.0) * jnp.clip(gy2 - gy1, 0.0)
    area_p = jnp.clip(px2 - px1, 0.0) * jnp.clip(py2 - py1, 0.0)
    iw = jnp.clip(jnp.minimum(gx2, px2) - jnp.maximum(gx1, px1), 0.0)
    ih = jnp.clip(jnp.minimum(gy2, py2) - jnp.maximum(gy1, py1), 0.0)
    inter = iw * ih
    union = area_g + area_p - inter
    ious = inter / jnp.clip(union, 1e-8)

    iota_c = jax.lax.broadcasted_iota(jnp.int32, (G, NUM_CLASSES), 1)
    onehot_lab = (iota_c == glab).astype(jnp.float32)
    ll = jax.lax.dot_general(
        onehot_lab, x_cls, (((1,), (1,)), ((), ())),
        preferred_element_type=jnp.float32)

    la = jnp.abs(ll)
    le = jnp.exp(-la)
    lL = jnp.log(1.0 + le)
    lmax0 = jnp.maximum(ll, 0.0)
    lnum = jnp.where(ll >= 0.0, 1.0, le)
    lp = lnum / (1.0 + le)

    sf = (ious - lp) * (ious - lp)
    ce = lmax0 - ll * ious + lL
    cost = ce * sf - 3.0 * jnp.log(ious + 1e-8)
    cost = jnp.where(is_in_gt & (valid_any > 0.0), cost, cost + 1e8)

    work = ious
    s10 = jnp.zeros((G, 1), jnp.float32)
    cum = jnp.zeros((G, 1), jnp.float32)
    for _ in range(TOPK):
        vmax = jnp.max(work, axis=1, keepdims=True)
        eq = work == vmax
        cnt = _dot(jnp.where(eq, 1.0, 0.0), ones_m)
        s10 = s10 + vmax * jnp.clip(float(TOPK) - cum, 0.0, cnt)
        cum = cum + cnt
        work = jnp.where(eq, -1.0, work)
    ks = jnp.clip(jnp.floor(s10), 1.0, float(M))

    work = cost
    match = jnp.zeros((G, M), jnp.float32)
    for i in range(TOPK):
        vmin = jnp.min(work, axis=1, keepdims=True)
        imin = jnp.min(jnp.where(work == vmin, iota_m, BIGI),
                       axis=1, keepdims=True)
        sel = iota_m == imin
        kmask = jnp.where(float(i) < ks, 1.0, 0.0)
        match = match + jnp.where(sel, kmask, 0.0)
        work = jnp.where(sel, REMOVED, work)

    ones_g = jnp.ones((1, G), jnp.float32)
    n_match = _dot(ones_g, match)
    cmin = jnp.min(cost, axis=0, keepdims=True)
    gmin = jnp.min(jnp.where(cost == cmin, iota_g, BIGI),
                   axis=0, keepdims=True)
    keep = (iota_g == gmin).astype(jnp.float32)
    match = jnp.where(n_match > 1.0, keep, match)
    fg = _dot(ones_g, match)

    x = x_cls
    a = jnp.abs(x)
    e = jnp.exp(-a)
    L = jnp.log(1.0 + e)
    num = jnp.where(x >= 0.0, 1.0, e)
    p = num / (1.0 + e)
    loss_bg = (1.0 - ALPHA) * (jnp.maximum(x, 0.0) + L) * p * p
    s_bg = _dot(loss_bg, jnp.ones((NUM_CLASSES, 1), jnp.float32))
    bg_sum = _dot(inv_mask, s_bg)

    q = 1.0 - lp
    l1 = ALPHA * (lmax0 - ll + lL) * q * q
    l0 = (1.0 - ALPHA) * (lmax0 + lL) * lp * lp
    corr = _dot(ones_g, match * (l1 - l0))
    cls_sum = bg_sum + jnp.sum(inv_mask * corr, axis=1, keepdims=True)

    bt = jax.lax.dot_general(gbox, match, (((0,), (0,)), ((), ())),
                             preferred_element_type=jnp.float32)
    tx1, ty1, tx2, ty2 = (bt[0:1, :], bt[1:2, :], bt[2:3, :], bt[3:4, :])
    a2 = jnp.clip(tx2 - tx1, 0.0) * jnp.clip(ty2 - ty1, 0.0)
    iw = jnp.clip(jnp.minimum(px2, tx2) - jnp.maximum(px1, tx1), 0.0)
    ih = jnp.clip(jnp.minimum(py2, ty2) - jnp.maximum(py1, ty1), 0.0)
    inter = iw * ih
    union = area_p + a2 - inter
    iou = inter / jnp.clip(union, 1e-8)
    cw = jnp.clip(jnp.maximum(px2, tx2) - jnp.minimum(px1, tx1), 0.0)
    ch = jnp.clip(jnp.maximum(py2, ty2) - jnp.minimum(py1, ty1), 0.0)
    carea = cw * ch
    gi = iou - (carea - union) / jnp.clip(carea, 1e-8)
    reg_sum = jnp.sum(fg * (1.0 - gi), axis=1, keepdims=True)

    cls_ref[0] = cls_sum
    reg_ref[0] = reg_sum
    npos_ref[0] = jnp.sum(fg, axis=1, keepdims=True)


@jax.jit
def kernel(pred_cls, pred_box, anchors, mask, gt_labels, gt_bboxes):
    B, M, C = pred_cls.shape
    G = gt_bboxes.shape[1]

    pbt = jnp.transpose(pred_box, (0, 2, 1))
    anc = jnp.transpose(anchors[:, :2], (1, 0))
    mask_f = mask.astype(jnp.float32).reshape(B, 1, M)
    glab = gt_labels.astype(jnp.int32).reshape(B, G, 1)

    out_sd = jax.ShapeDtypeStruct((B, 1, 1), jnp.float32)
    cls_s, reg_s, npos = pl.pallas_call(
        _ota_kernel,
        grid=(B,),
        in_specs=[
            pl.BlockSpec((1, M, C), lambda b: (b, 0, 0)),
            pl.BlockSpec((1, 4, M), lambda b: (b, 0, 0)),
            pl.BlockSpec((2, M), lambda b: (0, 0)),
            pl.BlockSpec((1, 1, M), lambda b: (b, 0, 0)),
            pl.BlockSpec((1, G, 1), lambda b: (b, 0, 0)),
            pl.BlockSpec((1, G, 4), lambda b: (b, 0, 0)),
        ],
        out_specs=[
            pl.BlockSpec((1, 1, 1), lambda b: (b, 0, 0)),
            pl.BlockSpec((1, 1, 1), lambda b: (b, 0, 0)),
            pl.BlockSpec((1, 1, 1), lambda b: (b, 0, 0)),
        ],
        out_shape=[out_sd, out_sd, out_sd],
    )(pred_cls, pbt, anc, mask_f, glab, gt_bboxes)

    num_fgs = jnp.maximum(jnp.sum(npos), 1.0)
    return jnp.sum(cls_s) / num_fgs, jnp.sum(reg_s) / num_fgs
